# Initial kernel scaffold; baseline (speedup 1.0000x reference)
import jax
import jax.numpy as jnp
from jax.experimental import pallas as pl


def kernel(embed, edge_index, W1, b1, W2, b2):
    raise NotImplementedError("write your pallas kernel here")


# speedup vs baseline: 2.7936x; 2.7936x over previous
import functools

import jax
import jax.numpy as jnp
from jax import lax
from jax.experimental import pallas as pl
from jax.experimental.pallas import tpu as pltpu
from jax.experimental.pallas import tpu_sc as plsc

N = 10000
D = 128
H = 64
E = 320000

NC = 2
NS = 16
NW = NC * NS
CHUNK = 128
NCHUNK = E // CHUNK
BASE_ITERS = NCHUNK // NW
EXTRA = NCHUNK - BASE_ITERS * NW
NPAD = 10112
ROWS_PER_TILE = NPAD // NS


def _proj_body(x_ref, w1a_ref, w1b_ref, b1_ref, a_ref, b_ref):
    x = x_ref[...]
    a_ref[...] = jnp.dot(x, w1a_ref[...], preferred_element_type=jnp.float32)
    b_ref[...] = (
        jnp.dot(x, w1b_ref[...], preferred_element_type=jnp.float32)
        + b1_ref[...]
    )


_proj = pl.pallas_call(
    _proj_body,
    out_shape=[
        jax.ShapeDtypeStruct((N, D), jnp.float32),
        jax.ShapeDtypeStruct((N, D), jnp.float32),
    ],
)


def _combine_body(p_ref, o_ref):
    o_ref[...] = p_ref[0, :N] + p_ref[1, :N]


_combine = pl.pallas_call(
    _combine_body,
    out_shape=jax.ShapeDtypeStruct((N, D), jnp.float32),
)


_mesh = plsc.VectorSubcoreMesh(core_axis_name="c", subcore_axis_name="s")


@functools.partial(
    pl.kernel,
    out_type=jax.ShapeDtypeStruct((NC, NPAD, D), jnp.float32),
    mesh=_mesh,
    compiler_params=pltpu.CompilerParams(needs_layout_passes=False),
    scratch_types=[
        pltpu.VMEM((CHUNK,), jnp.int32),
        pltpu.VMEM((CHUNK,), jnp.int32),
        pltpu.VMEM((CHUNK, D), jnp.float32),
        pltpu.VMEM((CHUNK, D), jnp.float32),
        pltpu.VMEM((CHUNK, D), jnp.float32),
        pltpu.VMEM((80,), jnp.float32),
        pltpu.SMEM((80,), jnp.float32),
        pltpu.VMEM_SHARED((NPAD, D), jnp.float32),
        pltpu.SemaphoreType.DMA,
        pltpu.SemaphoreType.DMA,
        pltpu.SemaphoreType.DMA,
    ],
)
def _sc_edge(src_hbm, dst_hbm, a_hbm, b_hbm, e_hbm, w2_hbm, out_hbm,
             src_v, dst_v, arows, brows, erows, w2_v, w2_s,
             acc, sem1, sem2, sem3):
    cid = lax.axis_index("c")
    sid = lax.axis_index("s")
    wid = sid * NC + cid

    zeros16 = jnp.zeros((16,), jnp.float32)

    def zrow(r, carry):
        for q in range(D // 16):
            erows[r, pl.ds(q * 16, 16)] = zeros16
        return carry

    lax.fori_loop(0, CHUNK, zrow, 0)
    for k in range(ROWS_PER_TILE // CHUNK):
        pltpu.sync_copy(
            erows, acc.at[pl.ds(sid * ROWS_PER_TILE + k * CHUNK, CHUNK)]
        )
    rem = ROWS_PER_TILE % CHUNK
    if rem:
        pltpu.sync_copy(
            erows.at[pl.ds(0, rem)],
            acc.at[pl.ds(sid * ROWS_PER_TILE
                         + (ROWS_PER_TILE // CHUNK) * CHUNK, rem)],
        )
    pltpu.sync_copy(w2_hbm, w2_v)
    for q in range(80 // 16):
        vq = w2_v[pl.ds(q * 16, 16)]
        for i in range(16):
            w2_s[q * 16 + i] = vq[i]
    plsc.subcore_barrier()

    niter = BASE_ITERS + jnp.where(wid < EXTRA, 1, 0)
    lane = lax.iota(jnp.int32, 16)
    b2s = w2_s[64]

    def chunk_body(i, carry):
        base = (i * NW + wid) * CHUNK
        pltpu.sync_copy(src_hbm.at[pl.ds(base, CHUNK)], src_v)
        pltpu.sync_copy(dst_hbm.at[pl.ds(base, CHUNK)], dst_v)
        cp1 = pltpu.async_copy(a_hbm.at[src_v], arows, sem1)
        cp2 = pltpu.async_copy(b_hbm.at[dst_v], brows, sem2)
        cp3 = pltpu.async_copy(e_hbm.at[src_v], erows, sem3)
        cp1.wait()
        cp2.wait()
        cp3.wait()

        def group_body(g, gc):
            rows = g * 16 + lane
            lvec = jnp.zeros((16,), jnp.float32)
            for j in range(H):
                colj = jnp.full((16,), j, jnp.int32)
                av = plsc.load_gather(arows, [rows, colj])
                bv = plsc.load_gather(brows, [rows, colj])
                lvec = lvec + jnp.maximum(av + bv, 0.0) * w2_s[j]
            mvec = 1.0 / (1.0 + jnp.exp(-(lvec + b2s)))
            for ii in range(16):
                e = g * 16 + ii
                m = mvec[ii]
                for q in range(D // 16):
                    erows[e, pl.ds(q * 16, 16)] = (
                        erows[e, pl.ds(q * 16, 16)] * m
                    )
            return gc

        lax.fori_loop(0, CHUNK // 16, group_body, 0)
        pltpu.sync_copy(erows, acc.at[dst_v], add=True)
        return carry

    lax.fori_loop(0, niter, chunk_body, 0)
    plsc.subcore_barrier()

    r0 = sid * ROWS_PER_TILE
    pltpu.sync_copy(acc.at[pl.ds(r0, ROWS_PER_TILE)],
                    out_hbm.at[cid, pl.ds(r0, ROWS_PER_TILE)])


@jax.jit
def kernel(embed, edge_index, W1, b1, W2, b2):
    pad = jnp.zeros((D, D - H), jnp.float32)
    w1a = jnp.concatenate([W1[:D], pad], axis=1)
    w1b = jnp.concatenate([W1[D:], pad], axis=1)
    b1p = jnp.concatenate([b1, jnp.zeros((D - H,), jnp.float32)])
    a, b = _proj(embed, w1a, w1b, b1p.reshape(1, D))
    w2full = jnp.concatenate(
        [W2[:, 0], b2, jnp.zeros((15,), jnp.float32)]
    )
    partial = _sc_edge(edge_index[0], edge_index[1], a, b, embed, w2full)
    return _combine(partial)


# speedup vs baseline: 3.4649x; 1.2403x over previous
import functools

import jax
import jax.numpy as jnp
from jax import lax
from jax.experimental import pallas as pl
from jax.experimental.pallas import tpu as pltpu
from jax.experimental.pallas import tpu_sc as plsc

N = 10000
D = 128
H = 64
E = 320000

NC = 2
NS = 16
NW = NC * NS
CHUNK = 64
EPW = E // NW
FULL = EPW // CHUNK
TAIL = EPW - FULL * CHUNK
EDGE_PAD = 2 * CHUNK
NPAD = 10112
ROWS_PER_TILE = NPAD // NS
DUMMY = N + 16


def _proj_body(x_ref, w1a_ref, w1b_ref, b1_ref, a_ref, b_ref):
    x = x_ref[...]
    a_ref[...] = jnp.dot(x, w1a_ref[...], preferred_element_type=jnp.float32)
    b_ref[...] = (
        jnp.dot(x, w1b_ref[...], preferred_element_type=jnp.float32)
        + b1_ref[...]
    )


_proj = pl.pallas_call(
    _proj_body,
    out_shape=[
        jax.ShapeDtypeStruct((N, D), jnp.float32),
        jax.ShapeDtypeStruct((N, D), jnp.float32),
    ],
)


def _combine_body(p_ref, o_ref):
    o_ref[...] = p_ref[0, :N] + p_ref[1, :N]


_combine = pl.pallas_call(
    _combine_body,
    out_shape=jax.ShapeDtypeStruct((N, D), jnp.float32),
)


_mesh = plsc.VectorSubcoreMesh(core_axis_name="c", subcore_axis_name="s")


@functools.partial(
    pl.kernel,
    out_type=jax.ShapeDtypeStruct((NC, NPAD, D), jnp.float32),
    mesh=_mesh,
    compiler_params=pltpu.CompilerParams(needs_layout_passes=False),
    scratch_types=[
        pltpu.VMEM((CHUNK,), jnp.int32),
        pltpu.VMEM((CHUNK,), jnp.int32),
        pltpu.VMEM((CHUNK, D), jnp.float32),
        pltpu.VMEM((CHUNK, D), jnp.float32),
        pltpu.VMEM((CHUNK, D), jnp.float32),
        pltpu.VMEM((CHUNK,), jnp.int32),
        pltpu.VMEM((CHUNK,), jnp.int32),
        pltpu.VMEM((CHUNK, D), jnp.float32),
        pltpu.VMEM((CHUNK, D), jnp.float32),
        pltpu.VMEM((CHUNK, D), jnp.float32),
        pltpu.VMEM((80,), jnp.float32),
        pltpu.SMEM((80,), jnp.float32),
        pltpu.VMEM_SHARED((NPAD, D), jnp.float32),
        pltpu.SemaphoreType.DMA,
        pltpu.SemaphoreType.DMA,
        pltpu.SemaphoreType.DMA,
        pltpu.SemaphoreType.DMA,
    ],
)
def _sc_edge(src_hbm, dst_hbm, a_hbm, b_hbm, e_hbm, w2_hbm, out_hbm,
             srcA, dstA, arowsA, browsA, erowsA,
             srcB, dstB, arowsB, browsB, erowsB,
             w2_v, w2_s, acc, semA, semB, semIA, semIB):
    cid = lax.axis_index("c")
    sid = lax.axis_index("s")
    wid = sid * NC + cid
    base_w = wid * EPW

    zeros16 = jnp.zeros((16,), jnp.float32)

    def zrow(r, carry):
        for q in range(D // 16):
            erowsA[r, pl.ds(q * 16, 16)] = zeros16
        return carry

    lax.fori_loop(0, CHUNK, zrow, 0)
    for k in range(ROWS_PER_TILE // CHUNK):
        pltpu.sync_copy(
            erowsA, acc.at[pl.ds(sid * ROWS_PER_TILE + k * CHUNK, CHUNK)]
        )
    rem = ROWS_PER_TILE % CHUNK
    if rem:
        pltpu.sync_copy(
            erowsA.at[pl.ds(0, rem)],
            acc.at[pl.ds(sid * ROWS_PER_TILE
                         + (ROWS_PER_TILE // CHUNK) * CHUNK, rem)],
        )
    pltpu.sync_copy(w2_hbm, w2_v)
    for q in range(80 // 16):
        vq = w2_v[pl.ds(q * 16, 16)]
        for i in range(16):
            w2_s[q * 16 + i] = vq[i]
    plsc.subcore_barrier()

    lane = lax.iota(jnp.int32, 16)
    b2s = w2_s[64]

    def idx_start(c, src_v, dst_v, sem):
        b = base_w + c * CHUNK
        c1 = pltpu.async_copy(src_hbm.at[pl.ds(b, CHUNK)], src_v, sem)
        c2 = pltpu.async_copy(dst_hbm.at[pl.ds(b, CHUNK)], dst_v, sem)
        return c1, c2

    def idx_wait(src_v, dst_v, sem):
        pltpu.make_async_copy(src_hbm.at[pl.ds(0, CHUNK)], src_v, sem).wait()
        pltpu.make_async_copy(dst_hbm.at[pl.ds(0, CHUNK)], dst_v, sem).wait()

    def gather_start(src_v, dst_v, arows, brows, erows, sem):
        pltpu.async_copy(a_hbm.at[src_v], arows, sem)
        pltpu.async_copy(b_hbm.at[dst_v], brows, sem)
        pltpu.async_copy(e_hbm.at[src_v], erows, sem)

    def gather_wait(src_v, dst_v, arows, brows, erows, sem):
        pltpu.make_async_copy(a_hbm.at[src_v], arows, sem).wait()
        pltpu.make_async_copy(b_hbm.at[dst_v], brows, sem).wait()
        pltpu.make_async_copy(e_hbm.at[src_v], erows, sem).wait()

    def compute(arows, brows, erows):
        def group_body(g, gc):
            rows = g * 16 + lane
            lvec = jnp.zeros((16,), jnp.float32)
            for j in range(H):
                colj = jnp.full((16,), j, jnp.int32)
                av = plsc.load_gather(arows, [rows, colj])
                bv = plsc.load_gather(brows, [rows, colj])
                lvec = lvec + jnp.maximum(av + bv, 0.0) * w2_s[j]
            mvec = 1.0 / (1.0 + jnp.exp(-(lvec + b2s)))
            for ii in range(16):
                m = mvec[ii]
                e = g * 16 + ii
                for q in range(D // 16):
                    erows[e, pl.ds(q * 16, 16)] = (
                        erows[e, pl.ds(q * 16, 16)] * m
                    )
            return gc

        lax.fori_loop(0, CHUNK // 16, group_body, 0)

    idx_start(0, srcA, dstA, semIA)
    idx_wait(srcA, dstA, semIA)
    gather_start(srcA, dstA, arowsA, browsA, erowsA, semA)
    idx_start(1, srcB, dstB, semIB)

    def pair_body(it, carry):
        c0 = 2 * it
        gather_wait(srcA, dstA, arowsA, browsA, erowsA, semA)
        idx_wait(srcB, dstB, semIB)
        gather_start(srcB, dstB, arowsB, browsB, erowsB, semB)
        compute(arowsA, browsA, erowsA)
        pltpu.sync_copy(erowsA, acc.at[dstA], add=True)
        idx_start(c0 + 2, srcA, dstA, semIA)
        gather_wait(srcB, dstB, arowsB, browsB, erowsB, semB)
        idx_wait(srcA, dstA, semIA)
        gather_start(srcA, dstA, arowsA, browsA, erowsA, semA)
        compute(arowsB, browsB, erowsB)
        pltpu.sync_copy(erowsB, acc.at[dstB], add=True)
        idx_start(c0 + 3, srcB, dstB, semIB)
        return carry

    lax.fori_loop(0, FULL // 2, pair_body, 0)

    gather_wait(srcA, dstA, arowsA, browsA, erowsA, semA)
    idx_wait(srcB, dstB, semIB)
    compute(arowsA, browsA, erowsA)
    dummy16 = jnp.full((16,), DUMMY, jnp.int32)
    for q in range(TAIL // 16, CHUNK // 16):
        dstA[pl.ds(q * 16, 16)] = dummy16
    pltpu.sync_copy(erowsA, acc.at[dstA], add=True)

    plsc.subcore_barrier()

    r0 = sid * ROWS_PER_TILE
    pltpu.sync_copy(acc.at[pl.ds(r0, ROWS_PER_TILE)],
                    out_hbm.at[cid, pl.ds(r0, ROWS_PER_TILE)])


@jax.jit
def kernel(embed, edge_index, W1, b1, W2, b2):
    pad = jnp.zeros((D, D - H), jnp.float32)
    w1a = jnp.concatenate([W1[:D], pad], axis=1)
    w1b = jnp.concatenate([W1[D:], pad], axis=1)
    b1p = jnp.concatenate([b1, jnp.zeros((D - H,), jnp.float32)])
    a, b = _proj(embed, w1a, w1b, b1p.reshape(1, D))
    w2full = jnp.concatenate(
        [W2[:, 0], b2, jnp.zeros((15,), jnp.float32)]
    )
    zpad = jnp.zeros((EDGE_PAD,), jnp.int32)
    srcp = jnp.concatenate([edge_index[0], zpad])
    dstp = jnp.concatenate([edge_index[1], zpad])
    partial = _sc_edge(srcp, dstp, a, b, embed, w2full)
    return _combine(partial)


# speedup vs baseline: 7.9023x; 2.2806x over previous
import functools

import jax
import jax.numpy as jnp
from jax import lax
from jax.experimental import pallas as pl
from jax.experimental.pallas import tpu as pltpu
from jax.experimental.pallas import tpu_sc as plsc

N = 10000
D = 128
H = 64
E = 320000

NC = 2
NS = 16
NW = NC * NS
CHUNK = 64
EPW = E // NW
FULL = EPW // CHUNK
TAIL = EPW - FULL * CHUNK
EDGE_PAD = 2 * CHUNK
NPAD = 10112
ROWS_PER_TILE = NPAD // NS
DUMMY = N + 16


def _proj_body(x_ref, w1a_ref, w1b_ref, b1_ref, a_ref, b_ref):
    x = x_ref[...]
    a_ref[...] = jnp.dot(x, w1a_ref[...], preferred_element_type=jnp.float32)
    b_ref[...] = (
        jnp.dot(x, w1b_ref[...], preferred_element_type=jnp.float32)
        + b1_ref[...]
    )


_proj = pl.pallas_call(
    _proj_body,
    out_shape=[
        jax.ShapeDtypeStruct((N, D), jnp.float32),
        jax.ShapeDtypeStruct((N, D), jnp.float32),
    ],
)


def _combine_body(p_ref, o_ref):
    o_ref[...] = p_ref[0, :N] + p_ref[1, :N]


_combine = pl.pallas_call(
    _combine_body,
    out_shape=jax.ShapeDtypeStruct((N, D), jnp.float32),
)


_mesh = plsc.VectorSubcoreMesh(core_axis_name="c", subcore_axis_name="s")


@functools.partial(
    pl.kernel,
    out_type=jax.ShapeDtypeStruct((NC, NPAD, D), jnp.float32),
    mesh=_mesh,
    compiler_params=pltpu.CompilerParams(needs_layout_passes=False),
    scratch_types=[
        pltpu.VMEM((CHUNK,), jnp.int32),
        pltpu.VMEM((CHUNK,), jnp.int32),
        pltpu.VMEM((CHUNK, D), jnp.float32),
        pltpu.VMEM((CHUNK, D), jnp.float32),
        pltpu.VMEM((CHUNK, D), jnp.float32),
        pltpu.VMEM((CHUNK,), jnp.int32),
        pltpu.VMEM((CHUNK,), jnp.int32),
        pltpu.VMEM((CHUNK, D), jnp.float32),
        pltpu.VMEM((CHUNK, D), jnp.float32),
        pltpu.VMEM((CHUNK, D), jnp.float32),
        pltpu.VMEM((16 * 17,), jnp.float32),
        pltpu.VMEM((80,), jnp.float32),
        pltpu.SMEM((80,), jnp.float32),
        pltpu.VMEM_SHARED((NPAD, D), jnp.float32),
        pltpu.SemaphoreType.DMA,
        pltpu.SemaphoreType.DMA,
        pltpu.SemaphoreType.DMA,
        pltpu.SemaphoreType.DMA,
    ],
)
def _sc_edge(src_hbm, dst_hbm, a_hbm, b_hbm, e_hbm, w2_hbm, out_hbm,
             srcA, dstA, arowsA, browsA, erowsA,
             srcB, dstB, arowsB, browsB, erowsB,
             tbuf, w2_v, w2_s, acc, semA, semB, semIA, semIB):
    cid = lax.axis_index("c")
    sid = lax.axis_index("s")
    wid = sid * NC + cid
    base_w = wid * EPW

    zeros16 = jnp.zeros((16,), jnp.float32)

    def zrow(r, carry):
        for q in range(D // 16):
            erowsA[r, pl.ds(q * 16, 16)] = zeros16
        return carry

    lax.fori_loop(0, CHUNK, zrow, 0)
    for k in range(ROWS_PER_TILE // CHUNK):
        pltpu.sync_copy(
            erowsA, acc.at[pl.ds(sid * ROWS_PER_TILE + k * CHUNK, CHUNK)]
        )
    rem = ROWS_PER_TILE % CHUNK
    if rem:
        pltpu.sync_copy(
            erowsA.at[pl.ds(0, rem)],
            acc.at[pl.ds(sid * ROWS_PER_TILE
                         + (ROWS_PER_TILE // CHUNK) * CHUNK, rem)],
        )
    pltpu.sync_copy(w2_hbm, w2_v)
    for q in range(80 // 16):
        vq = w2_v[pl.ds(q * 16, 16)]
        for i in range(16):
            w2_s[q * 16 + i] = vq[i]
    plsc.subcore_barrier()

    lane = lax.iota(jnp.int32, 16)
    b2s = w2_s[64]
    idx17 = lane * 17
    w2regs = [w2_v[pl.ds(q * 16, 16)] for q in range(H // 16)]

    def idx_start(c, src_v, dst_v, sem):
        b = base_w + c * CHUNK
        c1 = pltpu.async_copy(src_hbm.at[pl.ds(b, CHUNK)], src_v, sem)
        c2 = pltpu.async_copy(dst_hbm.at[pl.ds(b, CHUNK)], dst_v, sem)
        return c1, c2

    def idx_wait(src_v, dst_v, sem):
        pltpu.make_async_copy(src_hbm.at[pl.ds(0, CHUNK)], src_v, sem).wait()
        pltpu.make_async_copy(dst_hbm.at[pl.ds(0, CHUNK)], dst_v, sem).wait()

    def gather_start(src_v, dst_v, arows, brows, erows, sem):
        pltpu.async_copy(a_hbm.at[src_v], arows, sem)
        pltpu.async_copy(b_hbm.at[dst_v], brows, sem)
        pltpu.async_copy(e_hbm.at[src_v], erows, sem)

    def gather_wait(src_v, dst_v, arows, brows, erows, sem):
        pltpu.make_async_copy(a_hbm.at[src_v], arows, sem).wait()
        pltpu.make_async_copy(b_hbm.at[dst_v], brows, sem).wait()
        pltpu.make_async_copy(e_hbm.at[src_v], erows, sem).wait()

    def compute(arows, brows, erows):
        def group_body(g, gc):
            for ii in range(16):
                e = g * 16 + ii
                acc_v = None
                for q in range(H // 16):
                    av = arows[e, pl.ds(q * 16, 16)]
                    bv = brows[e, pl.ds(q * 16, 16)]
                    hv = jnp.maximum(av + bv, 0.0) * w2regs[q]
                    acc_v = hv if acc_v is None else acc_v + hv
                plsc.store_scatter(tbuf, [idx17 + ii], acc_v)
            lvec = jnp.full((16,), 0.0, jnp.float32)
            for j in range(16):
                lvec = lvec + plsc.load_gather(tbuf, [lane + j * 17])
            mvec = 1.0 / (1.0 + jnp.exp(-(lvec + b2s)))
            for ii in range(16):
                m = mvec[ii]
                e = g * 16 + ii
                for q in range(D // 16):
                    erows[e, pl.ds(q * 16, 16)] = (
                        erows[e, pl.ds(q * 16, 16)] * m
                    )
            return gc

        lax.fori_loop(0, CHUNK // 16, group_body, 0)

    idx_start(0, srcA, dstA, semIA)
    idx_wait(srcA, dstA, semIA)
    gather_start(srcA, dstA, arowsA, browsA, erowsA, semA)
    idx_start(1, srcB, dstB, semIB)

    def pair_body(it, carry):
        c0 = 2 * it
        gather_wait(srcA, dstA, arowsA, browsA, erowsA, semA)
        idx_wait(srcB, dstB, semIB)
        gather_start(srcB, dstB, arowsB, browsB, erowsB, semB)
        compute(arowsA, browsA, erowsA)
        pltpu.sync_copy(erowsA, acc.at[dstA], add=True)
        idx_start(c0 + 2, srcA, dstA, semIA)
        gather_wait(srcB, dstB, arowsB, browsB, erowsB, semB)
        idx_wait(srcA, dstA, semIA)
        gather_start(srcA, dstA, arowsA, browsA, erowsA, semA)
        compute(arowsB, browsB, erowsB)
        pltpu.sync_copy(erowsB, acc.at[dstB], add=True)
        idx_start(c0 + 3, srcB, dstB, semIB)
        return carry

    lax.fori_loop(0, FULL // 2, pair_body, 0)

    gather_wait(srcA, dstA, arowsA, browsA, erowsA, semA)
    idx_wait(srcB, dstB, semIB)
    compute(arowsA, browsA, erowsA)
    dummy16 = jnp.full((16,), DUMMY, jnp.int32)
    for q in range(TAIL // 16, CHUNK // 16):
        dstA[pl.ds(q * 16, 16)] = dummy16
    pltpu.sync_copy(erowsA, acc.at[dstA], add=True)

    plsc.subcore_barrier()

    r0 = sid * ROWS_PER_TILE
    pltpu.sync_copy(acc.at[pl.ds(r0, ROWS_PER_TILE)],
                    out_hbm.at[cid, pl.ds(r0, ROWS_PER_TILE)])


@jax.jit
def kernel(embed, edge_index, W1, b1, W2, b2):
    pad = jnp.zeros((D, D - H), jnp.float32)
    w1a = jnp.concatenate([W1[:D], pad], axis=1)
    w1b = jnp.concatenate([W1[D:], pad], axis=1)
    b1p = jnp.concatenate([b1, jnp.zeros((D - H,), jnp.float32)])
    a, b = _proj(embed, w1a, w1b, b1p.reshape(1, D))
    w2full = jnp.concatenate(
        [W2[:, 0], b2, jnp.zeros((15,), jnp.float32)]
    )
    zpad = jnp.zeros((EDGE_PAD,), jnp.int32)
    srcp = jnp.concatenate([edge_index[0], zpad])
    dstp = jnp.concatenate([edge_index[1], zpad])
    partial = _sc_edge(srcp, dstp, a, b, embed, w2full)
    return _combine(partial)
